# baseline (speedup 1.0000x reference)
import functools

import jax
import jax.numpy as jnp
from jax import lax
from jax.experimental import pallas as pl
from jax.experimental.pallas import tpu as pltpu
from jax.experimental.pallas import tpu_sc as plsc

_NC = 2
_NS = 16
_NW = _NC * _NS
_L = 16


def _worker_id():
    return lax.axis_index("s") * _NC + lax.axis_index("c")


def _splat(x):
    return jnp.broadcast_to(x, (_L,))


@functools.lru_cache(maxsize=None)
def _build_transpose(V, D):
    assert D == 64 and V % 2 == 0
    full_tiles = V // 128
    tail_w = V - full_tiles * 128
    per_w = (full_tiles + _NW - 1) // _NW
    nbuf = 4
    slots = ((per_w + nbuf - 1) // nbuf) * nbuf

    mesh = plsc.VectorSubcoreMesh(core_axis_name="c", subcore_axis_name="s")

    @functools.partial(
        pl.kernel,
        out_type=jax.ShapeDtypeStruct((V // 2, 2 * D), jnp.float32),
        mesh=mesh,
        compiler_params=pltpu.CompilerParams(
            use_tc_tiling_on_sc=True, needs_layout_passes=False),
        scratch_types=[
            *[pltpu.VMEM((D, 128), jnp.float32)] * nbuf,
            *[pltpu.VMEM((64, 128), jnp.float32)] * nbuf,
            *[pltpu.SemaphoreType.DMA] * nbuf,
            *[pltpu.SemaphoreType.DMA] * nbuf,
        ],
    )
    def ktr(tokT_hbm, tail2_hbm, out_hbm, *bufs):
        slab = bufs[:nbuf]
        trans = bufs[nbuf:2 * nbuf]
        lsem = bufs[2 * nbuf:3 * nbuf]
        ssem = bufs[3 * nbuf:]
        wid = _worker_id()

        lane = lax.iota(jnp.int32, _L)
        srow, scol = [], []
        for g in range(8):
            p = lane + g * _L
            srow.append(p // 2)
            scol.append((p % 2) * D)

        def vt_of(i):
            return i * _NW + wid

        def load(i, b, start):
            vt = vt_of(i)

            @pl.when(vt < full_tiles)
            def _():
                d = pltpu.make_async_copy(
                    tokT_hbm.at[:, pl.ds(vt * 128, 128)], slab[b], lsem[b])
                d.start() if start else d.wait()

        def store(i, b, start):
            vt = vt_of(i)

            @pl.when(vt < full_tiles)
            def _():
                d = pltpu.make_async_copy(
                    trans[b], out_hbm.at[pl.ds(vt * 64, 64), :], ssem[b])
                d.start() if start else d.wait()

        if tail_w:
            @pl.when(wid == 0)
            def _():
                pltpu.sync_copy(
                    tail2_hbm, out_hbm.at[pl.ds(full_tiles * 64, tail_w // 2)])

        for b in range(nbuf):
            load(b, b, start=True)

        def outer(o, carry):
            for b in range(nbuf):
                i = o * nbuf + b

                @pl.when((i >= nbuf) & (vt_of(i - nbuf) < full_tiles))
                def _():
                    store(i - nbuf, b, start=False)

                @pl.when(vt_of(i) < full_tiles)
                def _():
                    load(i, b, start=False)

                    @plsc.parallel_loop(0, D, unroll=8)
                    def _(dd):
                        for g in range(8):
                            vals = slab[b][dd, pl.ds(g * _L, _L)]
                            plsc.store_scatter(
                                trans[b], [srow[g], scol[g] + dd], vals)

                    store(i, b, start=True)

                    @pl.when(vt_of(i + nbuf) < full_tiles)
                    def _():
                        load(i + nbuf, b, start=True)
            return carry

        lax.fori_loop(0, slots // nbuf, outer, 0)
        for i in range(slots - nbuf, slots):
            b = i % nbuf

            @pl.when(vt_of(i) < full_tiles)
            def _():
                store(i, b, start=False)

    return ktr


@functools.lru_cache(maxsize=None)
def _build_gather(B, T, V, D):
    assert D == 64 and B % 128 == 0
    nbt = B // 128
    t_per_w = T * nbt // _NW
    assert t_per_w * _NW == T * nbt and t_per_w % 2 == 0
    nbuf = 2

    mesh = plsc.VectorSubcoreMesh(core_axis_name="c", subcore_axis_name="s")

    @functools.partial(
        pl.kernel,
        out_type=jax.ShapeDtypeStruct((T, D // 8, nbt, 8, 128), jnp.float32),
        mesh=mesh,
        compiler_params=pltpu.CompilerParams(
            use_tc_tiling_on_sc=False, needs_layout_passes=False),
        scratch_types=[
            pltpu.VMEM((T, D), jnp.float32),
            *[pltpu.VMEM((128,), jnp.int32)] * nbuf,
            *[pltpu.VMEM((128, D), jnp.float32)] * nbuf,
            *[pltpu.VMEM((D // 8, 8, 128), jnp.float32)] * nbuf,
            *[pltpu.SemaphoreType.DMA] * nbuf,
            *[pltpu.SemaphoreType.DMA] * nbuf,
            *[pltpu.SemaphoreType.DMA] * nbuf,
        ],
    )
    def kg(xT_hbm, tok_hbm, pos_hbm, out_hbm, pos_v, *bufs):
        idx = bufs[:nbuf]
        rows = bufs[nbuf:2 * nbuf]
        trans = bufs[2 * nbuf:3 * nbuf]
        isem = bufs[3 * nbuf:4 * nbuf]
        gsem = bufs[4 * nbuf:5 * nbuf]
        osem = bufs[5 * nbuf:]
        wid = _worker_id()
        bt = wid % nbt
        t0 = (wid // nbt) * t_per_w
        pltpu.sync_copy(pos_hbm, pos_v)
        lane = lax.iota(jnp.int32, _L)

        srow, smid = [], []
        for k in range(D // _L):
            p = lane + k * _L
            srow.append(p // 8)
            smid.append(p % 8)

        def idx_desc(i, b):
            return pltpu.make_async_copy(
                xT_hbm.at[t0 + i, pl.ds(bt * 128, 128)], idx[b], isem[b])

        def gather_desc(b):
            return pltpu.make_async_copy(
                tok_hbm.at[idx[b]], rows[b], gsem[b])

        def out_desc(i, b):
            return pltpu.make_async_copy(
                trans[b], out_hbm.at[t0 + i, :, bt, :, :], osem[b])

        idx_desc(0, 0).start()

        def step(i, b):
            idx_desc(i, b).wait()
            gather_desc(b).start()

            @pl.when(i + 1 < t_per_w)
            def _():
                idx_desc(i + 1, 1 - b).start()

            t = t0 + i
            pos4 = [pos_v[t, pl.ds(k * _L, _L)] for k in range(D // _L)]

            @pl.when(i >= nbuf)
            def _():
                out_desc(i - nbuf, b).wait()

            gather_desc(b).wait()

            @plsc.parallel_loop(0, 128, unroll=8)
            def _(tok):
                tv = _splat(tok)
                for k in range(D // _L):
                    vals = rows[b][tok, pl.ds(k * _L, _L)] + pos4[k]
                    plsc.store_scatter(
                        trans[b], [srow[k], smid[k], tv], vals)

            out_desc(i, b).start()

        def outer(o, carry):
            for b in range(nbuf):
                step(o * nbuf + b, b)
            return carry

        lax.fori_loop(0, t_per_w // nbuf, outer, 0)
        for i in range(t_per_w - nbuf, t_per_w):
            out_desc(i, i % nbuf).wait()

    return kg


def kernel(x, token_table, pos_table):
    B, T = x.shape
    V, D = token_table.shape
    full = (V // 128) * 128
    tail2 = token_table[full:].reshape((V - full) // 2, 2 * D)
    tok2 = _build_transpose(V, D)(token_table.T, tail2)
    tok_rm = tok2.reshape(V, D)
    out5 = _build_gather(B, T, V, D)(x.T, tok_rm, pos_table)
    return out5.transpose(2, 4, 0, 1, 3).reshape(B, T, D)


# speedup vs baseline: 1.4418x; 1.4418x over previous
import functools

import jax
import jax.numpy as jnp
from jax import lax
from jax.experimental import pallas as pl
from jax.experimental.pallas import tpu as pltpu
from jax.experimental.pallas import tpu_sc as plsc

_NUM_CORES = 2
_NUM_SUBCORES = 16
_NUM_WORKERS = _NUM_CORES * _NUM_SUBCORES
_LANES = 16
_NBUF = 4


@functools.lru_cache(maxsize=None)
def _build(B, T, V, D):
    N = B * T
    assert N % _NUM_WORKERS == 0
    rows_per_w = N // _NUM_WORKERS
    assert rows_per_w % T == 0
    nchunks = rows_per_w // T
    assert nchunks % _NBUF == 0
    lanes_per_row = D // _LANES

    mesh = plsc.VectorSubcoreMesh(core_axis_name="c", subcore_axis_name="s")

    @functools.partial(
        pl.kernel,
        out_type=jax.ShapeDtypeStruct((N, D), jnp.float32),
        mesh=mesh,
        compiler_params=pltpu.CompilerParams(use_tc_tiling_on_sc=False),
        scratch_types=[
            pltpu.VMEM((rows_per_w,), jnp.int32),
            pltpu.VMEM((T, D), jnp.float32),
            *[pltpu.VMEM((T, D), jnp.float32)] * _NBUF,
            *[pltpu.SemaphoreType.DMA] * _NBUF,
            *[pltpu.SemaphoreType.DMA] * _NBUF,
        ],
    )
    def emb(x_hbm, tok_hbm, pos_hbm, out_hbm, idx_v, pos_v, *bufs):
        rows = bufs[:_NBUF]
        gsem = bufs[_NBUF:2 * _NBUF]
        ssem = bufs[2 * _NBUF:]
        wid = lax.axis_index("s") * _NUM_CORES + lax.axis_index("c")
        base = wid * rows_per_w
        pltpu.sync_copy(x_hbm.at[pl.ds(base, rows_per_w)], idx_v)
        pltpu.sync_copy(pos_hbm, pos_v)

        def gather_desc(t, b):
            return pltpu.make_async_copy(
                tok_hbm.at[idx_v.at[pl.ds(t * T, T)]], rows[b], gsem[b]
            )

        def scatter_desc(t, b):
            return pltpu.make_async_copy(
                rows[b], out_hbm.at[pl.ds(base + t * T, T)], ssem[b]
            )

        gather_desc(0, 0).start()

        def outer(i, carry):
            for b in range(_NBUF):
                t = i * _NBUF + b
                nb = (b + 1) % _NBUF

                @pl.when(t >= _NBUF - 1)
                def _():
                    scatter_desc(t - (_NBUF - 1), nb).wait()

                @pl.when(t + 1 < nchunks)
                def _():
                    gather_desc(t + 1, nb).start()

                gather_desc(t, b).wait()

                @plsc.parallel_loop(0, T, unroll=8)
                def _(r):
                    for c in range(lanes_per_row):
                        sl = pl.ds(c * _LANES, _LANES)
                        plsc.addupdate(rows[b].at[r, sl], pos_v[r, sl])

                scatter_desc(t, b).start()
            return carry

        lax.fori_loop(0, nchunks // _NBUF, outer, 0)
        for t in range(nchunks - _NBUF + 1, nchunks):
            scatter_desc(t, t % _NBUF).wait()

    return emb


def kernel(x, token_table, pos_table):
    B, T = x.shape
    V, D = token_table.shape
    emb = _build(B, T, V, D)
    flat_idx = x.reshape(-1).astype(jnp.int32)
    out = emb(flat_idx, token_table, pos_table)
    return out.reshape(B, T, D)
